# baseline (speedup 1.0000x reference)
import jax
import jax.numpy as jnp
from jax.experimental import pallas as pl
from jax.experimental.pallas import tpu as pltpu

_R0 = 200
_NR = 592
_C0 = 384
_NC = 1152
_MSK_R0 = 192
_MSK_NR = 608


def _iy_rel(yo):
    return (8 * (yo + 256) - 1) // 10 - _R0


def _sy_rel(yo):
    y = yo + 256
    r = y % 10
    return 8 * (y // 10) + r - (1 if r >= 4 else 0) - (1 if r >= 8 else 0) - _R0


def _sx_rel(xo):
    xa = 512 + xo
    r = xa % 10
    return 8 * (xa // 10) + r - (1 if r >= 4 else 0) - (1 if r >= 8 else 0) - _C0


_FLW_WSTART = tuple(_sx_rel(128 * t) for t in range(10))


def _win_copy(hbm_ref, win_ref, sem, ch):
    return pltpu.make_async_copy(
        hbm_ref.at[ch, pl.ds(_R0, _NR), pl.ds(_C0, _NC)], win_ref, sem)


def _msk_copy(hbm_ref, win_ref, sem):
    return pltpu.make_async_copy(
        hbm_ref.at[pl.ds(_MSK_R0, _MSK_NR), pl.ds(_C0, _NC)], win_ref, sem)


def _body(a_hbm, b_hbm, f_hbm, m_hbm, oa_ref, ob_ref, of_ref, ov_ref,
          wa_ref, wb_ref, wf_ref, wm_ref, va_ref, vb_ref, vc_ref, sems):
    c = pl.program_id(0)
    lane = jax.lax.broadcasted_iota(jnp.int32, (720, 128), 1)
    sub = jax.lax.broadcasted_iota(jnp.int32, (8, _NC), 0)

    @pl.when(c == 0)
    def _():
        _win_copy(a_hbm, wa_ref.at[0], sems.at[0, 0], 0).start()
        _win_copy(b_hbm, wb_ref.at[0], sems.at[0, 1], 0).start()

    @pl.when(c < 2)
    def _():
        _win_copy(a_hbm, wa_ref.at[(c + 1) % 2], sems.at[(c + 1) % 2, 0], c + 1).start()
        _win_copy(b_hbm, wb_ref.at[(c + 1) % 2], sems.at[(c + 1) % 2, 1], c + 1).start()

    @pl.when(c == 2)
    def _():
        _win_copy(f_hbm, wf_ref.at[0], sems.at[0, 2], 0).start()
        _win_copy(f_hbm, wf_ref.at[1], sems.at[1, 2], 1).start()
        _msk_copy(m_hbm, wm_ref, sems.at[0, 3]).start()

    @pl.when(c < 3)
    def _():
        slot = c % 2
        _win_copy(a_hbm, wa_ref.at[slot], sems.at[slot, 0], c).wait()
        _win_copy(b_hbm, wb_ref.at[slot], sems.at[slot, 1], c).wait()
        vcache = {}
        for v in range(90):
            yo0 = 8 * v
            m = _iy_rel(yo0)
            if v % 5 not in vcache:
                q = 8 * (sub + yo0 + 256) - 1
                vcache[v % 5] = (q // 10 - _R0 - m,
                                 (q - 10 * (q // 10)).astype(jnp.float32) * 0.1)
            iyrel, fy = vcache[v % 5]
            wina = wa_ref[slot, m:m + 8, :]
            winb = wb_ref[slot, m:m + 8, :]
            a0 = jnp.take_along_axis(wina, iyrel, axis=0)
            b0 = jnp.take_along_axis(winb, iyrel, axis=0)
            a1 = jnp.take_along_axis(wina, iyrel + 1, axis=0)
            b1 = jnp.take_along_axis(winb, iyrel + 1, axis=0)
            va_ref[yo0:yo0 + 8, :] = a0 * (1.0 - fy) + a1 * fy
            vb_ref[yo0:yo0 + 8, :] = b0 * (1.0 - fy) + b1 * fy
        tcache = {}
        for t in range(10):
            s = (8 * (512 + 128 * t) - 1) // 10 - _C0
            if t % 5 not in tcache:
                xq = 8 * (512 + 128 * t + lane) - 1
                jabs = xq // 10
                tcache[t % 5] = (jabs - (_C0 + s),
                                 (xq - 10 * jabs).astype(jnp.float32) * 0.1)
            jrel, fx = tcache[t % 5]
            wina = va_ref[:, s:s + 128]
            winb = vb_ref[:, s:s + 128]
            a0 = jnp.take_along_axis(wina, jrel, axis=1)
            b0 = jnp.take_along_axis(winb, jrel, axis=1)
            a1 = jnp.take_along_axis(wina, jrel + 1, axis=1)
            b1 = jnp.take_along_axis(winb, jrel + 1, axis=1)
            oa_ref[0, :, 128 * t:128 * (t + 1)] = a0 * (1.0 - fx) + a1 * fx
            ob_ref[0, :, 128 * t:128 * (t + 1)] = b0 * (1.0 - fx) + b1 * fx

    @pl.when(c == 3)
    def _():
        _win_copy(f_hbm, wf_ref.at[0], sems.at[0, 2], 0).wait()
        _win_copy(f_hbm, wf_ref.at[1], sems.at[1, 2], 1).wait()
        _msk_copy(m_hbm, wm_ref, sems.at[0, 3]).wait()

        mrow = _R0 - _MSK_R0
        fvcache = {}
        for v in range(90):
            yo0 = 8 * v
            m = _sy_rel(yo0)
            if v % 5 not in fvcache:
                y = sub + yo0 + 256
                r = y % 10
                sy = (8 * (y // 10) + r - (r >= 4).astype(jnp.int32)
                      - (r >= 8).astype(jnp.int32) - _R0)
                fvcache[v % 5] = (sy - m,
                                  jnp.logical_and(r != 3, r != 7).astype(jnp.float32))
            idx, rowkeep = fvcache[v % 5]
            winf0 = wf_ref[0, m:m + 8, :]
            winf1 = wf_ref[1, m:m + 8, :]
            winm = wm_ref[m + mrow:m + mrow + 8, :].astype(jnp.float32)
            vmc = jnp.take_along_axis(winm, idx, axis=0) * rowkeep
            vc_ref[yo0:yo0 + 8, :] = vmc
            va_ref[yo0:yo0 + 8, :] = jnp.take_along_axis(winf0, idx, axis=0) * 1.25 * vmc
            vb_ref[yo0:yo0 + 8, :] = jnp.take_along_axis(winf1, idx, axis=0) * 1.25 * vmc
        ftcache = {}
        for t in range(10):
            w = _FLW_WSTART[t]
            if t % 5 not in ftcache:
                xa = 512 + 128 * t + lane
                r = xa % 10
                sx = (8 * (xa // 10) + r - (r >= 4).astype(jnp.int32)
                      - (r >= 8).astype(jnp.int32))
                cm = jnp.logical_and(r != 3, r != 7)
                ftcache[t % 5] = (sx - (_C0 + w), cm, cm.astype(jnp.float32))
            jrel, colmask, colf = ftcache[t % 5]
            g0 = jnp.take_along_axis(va_ref[:, w:w + 128], jrel, axis=1)
            g1 = jnp.take_along_axis(vb_ref[:, w:w + 128], jrel, axis=1)
            gm = jnp.take_along_axis(vc_ref[:, w:w + 128], jrel, axis=1)
            of_ref[0, :, 128 * t:128 * (t + 1)] = g0 * colf
            of_ref[1, :, 128 * t:128 * (t + 1)] = g1 * colf
            ov_ref[:, 128 * t:128 * (t + 1)] = jnp.logical_and(gm > 0.5, colmask)


def kernel(img1, img2, flow, valid_flow_mask):
    mk8 = valid_flow_mask.view(jnp.int8)
    o1, o2, fo, vo = pl.pallas_call(
        _body,
        grid=(4,),
        in_specs=[pl.BlockSpec(memory_space=pl.ANY)] * 4,
        out_specs=[
            pl.BlockSpec((1, 720, 1280), lambda c: (jnp.minimum(c, 2), 0, 0)),
            pl.BlockSpec((1, 720, 1280), lambda c: (jnp.minimum(c, 2), 0, 0)),
            pl.BlockSpec((2, 720, 1280), lambda c: (0, 0, 0)),
            pl.BlockSpec((720, 1280), lambda c: (0, 0)),
        ],
        out_shape=[
            jax.ShapeDtypeStruct((3, 720, 1280), jnp.float32),
            jax.ShapeDtypeStruct((3, 720, 1280), jnp.float32),
            jax.ShapeDtypeStruct((2, 720, 1280), jnp.float32),
            jax.ShapeDtypeStruct((720, 1280), jnp.bool_),
        ],
        scratch_shapes=[
            pltpu.VMEM((2, _NR, _NC), jnp.float32),
            pltpu.VMEM((2, _NR, _NC), jnp.float32),
            pltpu.VMEM((2, _NR, _NC), jnp.float32),
            pltpu.VMEM((_MSK_NR, _NC), jnp.int8),
            pltpu.VMEM((720, _NC), jnp.float32),
            pltpu.VMEM((720, _NC), jnp.float32),
            pltpu.VMEM((720, _NC), jnp.float32),
            pltpu.SemaphoreType.DMA((2, 4)),
        ],
    )(img1, img2, flow, mk8)
    return o1, o2, fo, vo


# speedup vs baseline: 1.0851x; 1.0851x over previous
import jax
import jax.numpy as jnp
from jax.experimental import pallas as pl
from jax.experimental.pallas import tpu as pltpu

_R0 = 200
_NR = 592
_C0 = 384
_NC = 1152
_MSK_R0 = 192
_MSK_NR = 608


def _iy_rel(yo):
    return (8 * (yo + 256) - 1) // 10 - _R0


def _sy_rel(yo):
    y = yo + 256
    r = y % 10
    return 8 * (y // 10) + r - (1 if r >= 4 else 0) - (1 if r >= 8 else 0) - _R0


def _sx_rel(xo):
    xa = 512 + xo
    r = xa % 10
    return 8 * (xa // 10) + r - (1 if r >= 4 else 0) - (1 if r >= 8 else 0) - _C0


_FLW_WSTART = tuple(_sx_rel(128 * t) for t in range(10))


def _win_copy(hbm_ref, win_ref, sem, ch):
    return pltpu.make_async_copy(
        hbm_ref.at[ch, pl.ds(_R0, _NR), pl.ds(_C0, _NC)], win_ref, sem)


def _msk_copy(hbm_ref, win_ref, sem):
    return pltpu.make_async_copy(
        hbm_ref.at[pl.ds(_MSK_R0, _MSK_NR), pl.ds(_C0, _NC)], win_ref, sem)


def _body(a_hbm, b_hbm, f_hbm, m_hbm, oa_ref, ob_ref, of_ref, ov_ref,
          wa_ref, wb_ref, wf_ref, wm_ref, va_ref, vb_ref, vc_ref, sems):
    c = pl.program_id(0)
    lane = jax.lax.broadcasted_iota(jnp.int32, (720, 128), 1)
    sub = jax.lax.broadcasted_iota(jnp.int32, (8, _NC), 0)

    @pl.when(c == 0)
    def _():
        _win_copy(a_hbm, wa_ref.at[0], sems.at[0, 0], 0).start()
        _win_copy(b_hbm, wb_ref.at[0], sems.at[0, 1], 0).start()

    @pl.when(c < 2)
    def _():
        _win_copy(a_hbm, wa_ref.at[(c + 1) % 2], sems.at[(c + 1) % 2, 0], c + 1).start()
        _win_copy(b_hbm, wb_ref.at[(c + 1) % 2], sems.at[(c + 1) % 2, 1], c + 1).start()

    @pl.when(c == 2)
    def _():
        _win_copy(f_hbm, wf_ref.at[0], sems.at[0, 2], 0).start()
        _win_copy(f_hbm, wf_ref.at[1], sems.at[1, 2], 1).start()
        _msk_copy(m_hbm, wm_ref, sems.at[0, 3]).start()

    @pl.when(c < 3)
    def _():
        slot = c % 2
        _win_copy(a_hbm, wa_ref.at[slot], sems.at[slot, 0], c).wait()
        _win_copy(b_hbm, wb_ref.at[slot], sems.at[slot, 1], c).wait()
        for v in range(90):
            yo0 = 8 * v
            m = _iy_rel(yo0)
            q = 8 * (sub + yo0 + 256) - 1
            iyrel = q // 10 - _R0 - m
            fy = (q - 10 * (q // 10)).astype(jnp.float32) * 0.1
            wina = wa_ref[slot, m:m + 8, :]
            winb = wb_ref[slot, m:m + 8, :]
            a0 = jnp.take_along_axis(wina, iyrel, axis=0)
            b0 = jnp.take_along_axis(winb, iyrel, axis=0)
            a1 = jnp.take_along_axis(wina, iyrel + 1, axis=0)
            b1 = jnp.take_along_axis(winb, iyrel + 1, axis=0)
            va_ref[yo0:yo0 + 8, :] = a0 * (1.0 - fy) + a1 * fy
            vb_ref[yo0:yo0 + 8, :] = b0 * (1.0 - fy) + b1 * fy
        for t in range(10):
            s = (8 * (512 + 128 * t) - 1) // 10 - _C0
            xq = 8 * (512 + 128 * t + lane) - 1
            jabs = xq // 10
            jrel = jabs - (_C0 + s)
            fx = (xq - 10 * jabs).astype(jnp.float32) * 0.1
            wina = va_ref[:, s:s + 128]
            winb = vb_ref[:, s:s + 128]
            a0 = jnp.take_along_axis(wina, jrel, axis=1)
            b0 = jnp.take_along_axis(winb, jrel, axis=1)
            a1 = jnp.take_along_axis(wina, jrel + 1, axis=1)
            b1 = jnp.take_along_axis(winb, jrel + 1, axis=1)
            oa_ref[0, :, 128 * t:128 * (t + 1)] = a0 * (1.0 - fx) + a1 * fx
            ob_ref[0, :, 128 * t:128 * (t + 1)] = b0 * (1.0 - fx) + b1 * fx

    @pl.when(c == 3)
    def _():
        _win_copy(f_hbm, wf_ref.at[0], sems.at[0, 2], 0).wait()
        _win_copy(f_hbm, wf_ref.at[1], sems.at[1, 2], 1).wait()
        _msk_copy(m_hbm, wm_ref, sems.at[0, 3]).wait()

        mrow = _R0 - _MSK_R0
        for v in range(90):
            yo0 = 8 * v
            m = _sy_rel(yo0)
            y = sub + yo0 + 256
            r = y % 10
            sy = (8 * (y // 10) + r - (r >= 4).astype(jnp.int32)
                  - (r >= 8).astype(jnp.int32) - _R0)
            idx = sy - m
            rowkeep = jnp.logical_and(r != 3, r != 7).astype(jnp.float32)
            winf0 = wf_ref[0, m:m + 8, :]
            winf1 = wf_ref[1, m:m + 8, :]
            winm = wm_ref[m + mrow:m + mrow + 8, :].astype(jnp.float32)
            vmc = jnp.take_along_axis(winm, idx, axis=0) * rowkeep
            vc_ref[yo0:yo0 + 8, :] = vmc
            va_ref[yo0:yo0 + 8, :] = jnp.take_along_axis(winf0, idx, axis=0) * 1.25 * vmc
            vb_ref[yo0:yo0 + 8, :] = jnp.take_along_axis(winf1, idx, axis=0) * 1.25 * vmc
        for t in range(10):
            w = _FLW_WSTART[t]
            xa = 512 + 128 * t + lane
            r = xa % 10
            sx = (8 * (xa // 10) + r - (r >= 4).astype(jnp.int32)
                  - (r >= 8).astype(jnp.int32))
            jrel = sx - (_C0 + w)
            colmask = jnp.logical_and(r != 3, r != 7)
            colf = colmask.astype(jnp.float32)
            g0 = jnp.take_along_axis(va_ref[:, w:w + 128], jrel, axis=1)
            g1 = jnp.take_along_axis(vb_ref[:, w:w + 128], jrel, axis=1)
            gm = jnp.take_along_axis(vc_ref[:, w:w + 128], jrel, axis=1)
            of_ref[0, :, 128 * t:128 * (t + 1)] = g0 * colf
            of_ref[1, :, 128 * t:128 * (t + 1)] = g1 * colf
            ov_ref[:, 128 * t:128 * (t + 1)] = jnp.logical_and(gm > 0.5, colmask)


def kernel(img1, img2, flow, valid_flow_mask):
    mk8 = valid_flow_mask.view(jnp.int8)
    o1, o2, fo, vo = pl.pallas_call(
        _body,
        grid=(4,),
        in_specs=[pl.BlockSpec(memory_space=pl.ANY)] * 4,
        out_specs=[
            pl.BlockSpec((1, 720, 1280), lambda c: (jnp.minimum(c, 2), 0, 0)),
            pl.BlockSpec((1, 720, 1280), lambda c: (jnp.minimum(c, 2), 0, 0)),
            pl.BlockSpec((2, 720, 1280), lambda c: (0, 0, 0)),
            pl.BlockSpec((720, 1280), lambda c: (0, 0)),
        ],
        out_shape=[
            jax.ShapeDtypeStruct((3, 720, 1280), jnp.float32),
            jax.ShapeDtypeStruct((3, 720, 1280), jnp.float32),
            jax.ShapeDtypeStruct((2, 720, 1280), jnp.float32),
            jax.ShapeDtypeStruct((720, 1280), jnp.bool_),
        ],
        scratch_shapes=[
            pltpu.VMEM((2, _NR, _NC), jnp.float32),
            pltpu.VMEM((2, _NR, _NC), jnp.float32),
            pltpu.VMEM((2, _NR, _NC), jnp.float32),
            pltpu.VMEM((_MSK_NR, _NC), jnp.int8),
            pltpu.VMEM((720, _NC), jnp.float32),
            pltpu.VMEM((720, _NC), jnp.float32),
            pltpu.VMEM((720, _NC), jnp.float32),
            pltpu.SemaphoreType.DMA((2, 4)),
        ],
    )(img1, img2, flow, mk8)
    return o1, o2, fo, vo
